# baseline (speedup 1.0000x reference)
import functools

import jax
import jax.numpy as jnp
from jax import lax
from jax.experimental import pallas as pl
from jax.experimental.pallas import tpu as pltpu
from jax.experimental.pallas import tpu_sc as plsc

_INFO = plsc.get_sparse_core_info()
_NC, _NS, _L = _INFO.num_cores, _INFO.num_subcores, _INFO.num_lanes
_NW = _NC * _NS

_B = 16384
_D = 128
_B_PER_W = _B // _NW
_CHUNK = 128
_NCHUNK = _B_PER_W // _CHUNK


def _gather_body(labels_hbm, table_hbm, out_hbm, idx_v, rows_v, gsems, ssems):
    wid = lax.axis_index("s") * _NC + lax.axis_index("c")
    base = wid * _B_PER_W
    pltpu.sync_copy(labels_hbm.at[pl.ds(wid * _NCHUNK, _NCHUNK)], idx_v)
    gathers = [
        pltpu.async_copy(
            table_hbm.at[idx_v.at[j]],
            rows_v.at[pl.ds(j * _CHUNK, _CHUNK)],
            gsems.at[j],
        )
        for j in range(_NCHUNK)
    ]
    stores = []
    for j in range(_NCHUNK):
        gathers[j].wait()
        stores.append(
            pltpu.async_copy(
                rows_v.at[pl.ds(j * _CHUNK, _CHUNK)],
                out_hbm.at[pl.ds(base + j * _CHUNK, _CHUNK)],
                ssems.at[j],
            )
        )
    for s in stores:
        s.wait()


@jax.jit
def _embed(labels2d, table):
    mesh = plsc.VectorSubcoreMesh(core_axis_name="c", subcore_axis_name="s")
    run = pl.kernel(
        _gather_body,
        out_type=jax.ShapeDtypeStruct((_B, _D), jnp.float32),
        mesh=mesh,
        scratch_types=[
            pltpu.VMEM((_NCHUNK, _CHUNK), jnp.int32),
            pltpu.VMEM((_B_PER_W, _D), jnp.float32),
            pltpu.SemaphoreType.DMA((_NCHUNK,)),
            pltpu.SemaphoreType.DMA((_NCHUNK,)),
        ],
    )
    return run(labels2d, table)


def kernel(labels, train, table):
    del train
    labels2d = labels.astype(jnp.int32).reshape(_B // _CHUNK, _CHUNK)
    return _embed(labels2d, jnp.asarray(table, jnp.float32))


# speedup vs baseline: 1.0035x; 1.0035x over previous
import jax
import jax.numpy as jnp
from jax import lax
from jax.experimental import pallas as pl
from jax.experimental.pallas import tpu as pltpu
from jax.experimental.pallas import tpu_sc as plsc

_INFO = plsc.get_sparse_core_info()
_NC, _NS, _L = _INFO.num_cores, _INFO.num_subcores, _INFO.num_lanes
_NW = _NC * _NS

_B = 16384
_D = 128
_B_PER_W = _B // _NW
_CHUNK = 128
_NCHUNK = _B_PER_W // _CHUNK


def _gather_body(labels_hbm, table_hbm, out_hbm, idx_v, rows_v, isems, gsems):
    wid = lax.axis_index("s") * _NC + lax.axis_index("c")
    base = wid * _B_PER_W
    icopies = [
        pltpu.async_copy(
            labels_hbm.at[pl.ds(base + j * _CHUNK, _CHUNK)],
            idx_v.at[j],
            isems.at[j],
        )
        for j in range(_NCHUNK)
    ]
    gathers = []
    for j in range(_NCHUNK):
        icopies[j].wait()
        gathers.append(
            pltpu.async_copy(
                table_hbm.at[idx_v.at[j]],
                rows_v.at[pl.ds(j * _CHUNK, _CHUNK)],
                gsems.at[j],
            )
        )
    for g in gathers:
        g.wait()
    pltpu.sync_copy(rows_v, out_hbm.at[pl.ds(base, _B_PER_W)])


@jax.jit
def _embed(labels, table):
    mesh = plsc.VectorSubcoreMesh(core_axis_name="c", subcore_axis_name="s")
    run = pl.kernel(
        _gather_body,
        out_type=jax.ShapeDtypeStruct((_B, _D), jnp.float32),
        mesh=mesh,
        scratch_types=[
            pltpu.VMEM((_NCHUNK, _CHUNK), jnp.int32),
            pltpu.VMEM((_B_PER_W, _D), jnp.float32),
            pltpu.SemaphoreType.DMA((_NCHUNK,)),
            pltpu.SemaphoreType.DMA((_NCHUNK,)),
        ],
    )
    return run(labels, table)


def kernel(labels, train, table):
    del train
    return _embed(labels.astype(jnp.int32), table)


# speedup vs baseline: 1.1309x; 1.1269x over previous
import jax
import jax.numpy as jnp
from jax import lax
from jax.experimental import pallas as pl
from jax.experimental.pallas import tpu as pltpu
from jax.experimental.pallas import tpu_sc as plsc

_INFO = plsc.get_sparse_core_info()
_NC, _NS, _L = _INFO.num_cores, _INFO.num_subcores, _INFO.num_lanes
_NW = _NC * _NS

_B = 16384
_D = 128
_V = 1001
_B_PER_W = _B // _NW
_CHUNK = 128
_NCHUNK = _B_PER_W // _CHUNK

_ROWS_PER_SUB = 64


def _gather_body(labels_hbm, table_hbm, out_hbm, idx_v, rows_v, tab_s, isem, gsems):
    cid = lax.axis_index("c")
    sid = lax.axis_index("s")
    wid = sid * _NC + cid
    base = wid * _B_PER_W

    icopy = pltpu.async_copy(
        labels_hbm.at[pl.ds(wid * _NCHUNK, _NCHUNK)], idx_v, isem
    )

    for k in range(_NS):
        start = k * _ROWS_PER_SUB
        n = min(_ROWS_PER_SUB, _V - start)
        @pl.when(sid == k)
        def _():
            pltpu.sync_copy(
                table_hbm.at[pl.ds(start, n)], tab_s.at[pl.ds(start, n)]
            )
    plsc.subcore_barrier()

    icopy.wait()
    gathers = [
        pltpu.async_copy(
            tab_s.at[idx_v.at[j]],
            rows_v.at[pl.ds(j * _CHUNK, _CHUNK)],
            gsems.at[j],
        )
        for j in range(_NCHUNK)
    ]
    for g in gathers:
        g.wait()
    pltpu.sync_copy(rows_v, out_hbm.at[pl.ds(base, _B_PER_W)])


@jax.jit
def _embed(labels2d, table):
    mesh = plsc.VectorSubcoreMesh(core_axis_name="c", subcore_axis_name="s")
    run = pl.kernel(
        _gather_body,
        out_type=jax.ShapeDtypeStruct((_B, _D), jnp.float32),
        mesh=mesh,
        scratch_types=[
            pltpu.VMEM((_NCHUNK, _CHUNK), jnp.int32),
            pltpu.VMEM((_B_PER_W, _D), jnp.float32),
            pltpu.VMEM_SHARED((_V, _D), jnp.float32),
            pltpu.SemaphoreType.DMA,
            pltpu.SemaphoreType.DMA((_NCHUNK,)),
        ],
    )
    return run(labels2d, table)


def kernel(labels, train, table):
    del train
    labels2d = labels.astype(jnp.int32).reshape(_B // _CHUNK, _CHUNK)
    return _embed(labels2d, table)


# speedup vs baseline: 1.1664x; 1.0314x over previous
import jax
import jax.numpy as jnp
from jax import lax
from jax.experimental import pallas as pl
from jax.experimental.pallas import tpu as pltpu
from jax.experimental.pallas import tpu_sc as plsc

_INFO = plsc.get_sparse_core_info()
_NC, _NS, _L = _INFO.num_cores, _INFO.num_subcores, _INFO.num_lanes
_NW = _NC * _NS

_B = 16384
_D = 128
_V = 1001
_B_PER_W = _B // _NW
_CHUNK = 128
_NCHUNK = _B_PER_W // _CHUNK

_ROWS_PER_SUB = 64


def _gather_body(labels_hbm, table_hbm, out_hbm, idx_v, rows_v, tab_s, isem, gsems, ssems):
    cid = lax.axis_index("c")
    sid = lax.axis_index("s")
    wid = sid * _NC + cid
    base = wid * _B_PER_W

    icopy = pltpu.async_copy(
        labels_hbm.at[pl.ds(wid * _NCHUNK, _NCHUNK)], idx_v, isem
    )

    for k in range(_NS):
        start = k * _ROWS_PER_SUB
        n = min(_ROWS_PER_SUB, _V - start)
        @pl.when(sid == k)
        def _():
            pltpu.sync_copy(
                table_hbm.at[pl.ds(start, n)], tab_s.at[pl.ds(start, n)]
            )
    plsc.subcore_barrier()

    icopy.wait()
    gathers = [
        pltpu.async_copy(
            tab_s.at[idx_v.at[j]],
            rows_v.at[pl.ds(j * _CHUNK, _CHUNK)],
            gsems.at[j],
        )
        for j in range(_NCHUNK)
    ]
    stores = []
    for j in range(_NCHUNK):
        gathers[j].wait()
        stores.append(
            pltpu.async_copy(
                rows_v.at[pl.ds(j * _CHUNK, _CHUNK)],
                out_hbm.at[pl.ds(base + j * _CHUNK, _CHUNK)],
                ssems.at[j],
            )
        )
    for s in stores:
        s.wait()


@jax.jit
def _embed(labels2d, table):
    mesh = plsc.VectorSubcoreMesh(core_axis_name="c", subcore_axis_name="s")
    run = pl.kernel(
        _gather_body,
        out_type=jax.ShapeDtypeStruct((_B, _D), jnp.float32),
        mesh=mesh,
        scratch_types=[
            pltpu.VMEM((_NCHUNK, _CHUNK), jnp.int32),
            pltpu.VMEM((_B_PER_W, _D), jnp.float32),
            pltpu.VMEM_SHARED((_V, _D), jnp.float32),
            pltpu.SemaphoreType.DMA,
            pltpu.SemaphoreType.DMA((_NCHUNK,)),
            pltpu.SemaphoreType.DMA((_NCHUNK,)),
        ],
    )
    return run(labels2d, table)


def kernel(labels, train, table):
    del train
    labels2d = labels.astype(jnp.int32).reshape(_B // _CHUNK, _CHUNK)
    return _embed(labels2d, table)


# speedup vs baseline: 1.1767x; 1.0088x over previous
import jax
import jax.numpy as jnp
from jax import lax
from jax.experimental import pallas as pl
from jax.experimental.pallas import tpu as pltpu
from jax.experimental.pallas import tpu_sc as plsc

_INFO = plsc.get_sparse_core_info()
_NC, _NS, _L = _INFO.num_cores, _INFO.num_subcores, _INFO.num_lanes
_NW = _NC * _NS

_B = 16384
_D = 128
_V = 1001
_B_PER_W = _B // _NW
_CHUNK = 128
_NCHUNK = _B_PER_W // _CHUNK

_ROWS_PER_SUB = 64


def _gather_body(labels_hbm, table_hbm, out_hbm, idx_v, rows_v, tab_s, isem, gsems, ssems):
    cid = lax.axis_index("c")
    sid = lax.axis_index("s")
    wid = sid * _NC + cid
    base = wid * _B_PER_W

    icopy = pltpu.async_copy(
        labels_hbm.at[pl.ds(wid * _NCHUNK, _NCHUNK)], idx_v, isem
    )

    start = pl.multiple_of(sid * _ROWS_PER_SUB, _ROWS_PER_SUB)

    @pl.when(sid < _NS - 1)
    def _():
        pltpu.sync_copy(
            table_hbm.at[pl.ds(start, _ROWS_PER_SUB)],
            tab_s.at[pl.ds(start, _ROWS_PER_SUB)],
        )

    @pl.when(sid == _NS - 1)
    def _():
        tail = (_NS - 1) * _ROWS_PER_SUB
        pltpu.sync_copy(
            table_hbm.at[pl.ds(tail, _V - tail)],
            tab_s.at[pl.ds(tail, _V - tail)],
        )

    plsc.subcore_barrier()

    icopy.wait()
    gathers = [
        pltpu.async_copy(
            tab_s.at[idx_v.at[j]],
            rows_v.at[pl.ds(j * _CHUNK, _CHUNK)],
            gsems.at[j],
        )
        for j in range(_NCHUNK)
    ]
    stores = []
    for j in range(_NCHUNK):
        gathers[j].wait()
        stores.append(
            pltpu.async_copy(
                rows_v.at[pl.ds(j * _CHUNK, _CHUNK)],
                out_hbm.at[pl.ds(base + j * _CHUNK, _CHUNK)],
                ssems.at[j],
            )
        )
    for s in stores:
        s.wait()


@jax.jit
def _embed(labels2d, table):
    mesh = plsc.VectorSubcoreMesh(core_axis_name="c", subcore_axis_name="s")
    run = pl.kernel(
        _gather_body,
        out_type=jax.ShapeDtypeStruct((_B, _D), jnp.float32),
        mesh=mesh,
        scratch_types=[
            pltpu.VMEM((_NCHUNK, _CHUNK), jnp.int32),
            pltpu.VMEM((_B_PER_W, _D), jnp.float32),
            pltpu.VMEM_SHARED((_V, _D), jnp.float32),
            pltpu.SemaphoreType.DMA,
            pltpu.SemaphoreType.DMA((_NCHUNK,)),
            pltpu.SemaphoreType.DMA((_NCHUNK,)),
        ],
    )
    return run(labels2d, table)


def kernel(labels, train, table):
    del train
    labels2d = labels.astype(jnp.int32).reshape(_B // _CHUNK, _CHUNK)
    return _embed(labels2d, table)
